# baseline (speedup 1.0000x reference)
import functools

import jax
import jax.numpy as jnp
from jax import lax
from jax.experimental import pallas as pl
from jax.experimental.pallas import tpu as pltpu
from jax.experimental.pallas import tpu_sc as plsc

_W = 7
_K = 16
_NCODES = 1 << _W


@functools.lru_cache(maxsize=None)
def _build_sc_call(B: int):
    info = plsc.get_sparse_core_info()
    NC, NS, L = info.num_cores, info.num_subcores, info.num_lanes
    NW = NC * NS
    assert B % (NW * L) == 0, B
    s_per_w = B // NW

    ch = 4096
    while s_per_w % ch:
        ch //= 2
    nchunk = s_per_w // ch

    mesh = plsc.VectorSubcoreMesh(core_axis_name="c", subcore_axis_name="s")

    @functools.partial(
        pl.kernel,
        mesh=mesh,
        out_type=jax.ShapeDtypeStruct((_W * B,), jnp.float32),
        compiler_params=pltpu.CompilerParams(needs_layout_passes=False, disable_bounds_checks=True),
        scratch_types=[
            pltpu.VMEM((128,), jnp.float32),
            pltpu.VMEM((_W * _NCODES,), jnp.float32),
            pltpu.VMEM((3 * _NCODES,), jnp.int32),
            pltpu.VMEM((2 * _W * ch,), jnp.int32),
            pltpu.VMEM((2 * _W * ch,), jnp.float32),
            pltpu.SemaphoreType.DMA((2,)),
            pltpu.SemaphoreType.DMA((2,)),
        ],
    )
    def sc_call(hd_hbm, c_hbm, out_hbm, c_v, lut_v, plut_v, in_v, out_v, sin, sout):
        iota = lax.iota(jnp.int32, L)
        lane_lt7 = iota < _W
        col7 = jnp.minimum(iota, _W - 1)

        wid = lax.axis_index("s") * NC + lax.axis_index("c")
        tile_base = wid * s_per_w

        def in_copies(ci, buf):
            off = pl.multiple_of(tile_base + ci * ch, 8)
            return [
                pltpu.make_async_copy(
                    hd_hbm.at[pl.ds(j * B + off, ch)],
                    in_v.at[pl.ds((buf * _W + j) * ch, ch)],
                    sin.at[buf],
                )
                for j in range(_W)
            ]

        def out_copies(ci, buf):
            off = pl.multiple_of(tile_base + ci * ch, 8)
            return [
                pltpu.make_async_copy(
                    out_v.at[pl.ds((buf * _W + j) * ch, ch)],
                    out_hbm.at[pl.ds(j * B + off, ch)],
                    sout.at[buf],
                )
                for j in range(_W)
            ]

        def start_all(copies):
            for cp in copies:
                cp.start()

        def wait_all(copies):
            for cp in copies:
                cp.wait()

        start_all(in_copies(0, 0))
        if nchunk > 1:
            start_all(in_copies(1, 1))

        pltpu.sync_copy(c_hbm, c_v.at[pl.ds(0, _K * _W)])

        ccols = [
            plsc.load_gather(c_v, [iota * _W + j]).astype(jnp.int32)
            for j in range(_W)
        ]

        def build_body(w, carry):
            dist = jnp.zeros((L,), jnp.int32)
            for j in range(_W):
                bit = (w >> (_W - 1 - j)) & 1
                dist = dist + jnp.where(ccols[j] == bit, 1, 0).astype(jnp.int32)
            score = dist * _K + (_K - 1 - iota)
            best = jnp.max(score)
            bestk = (_K - 1) - (best & (_K - 1))
            row = plsc.load_gather(c_v, [bestk * _W + col7], mask=lane_lt7)
            plsc.store_scatter(lut_v, [col7 * _NCODES + w], row, mask=lane_lt7)
            return carry

        lax.fori_loop(0, _NCODES, build_body, 0)

        for p in range(3):
            for cb in range(_NCODES // L):
                a = lut_v[pl.ds((2 * p) * _NCODES + cb * L, L)]
                b = lut_v[pl.ds((2 * p + 1) * _NCODES + cb * L, L)]
                packed = plsc.bitcast(
                    plsc.pack(a, b, format=plsc.PackFormat.INTERLEAVED),
                    jnp.int32,
                )
                plut_v[pl.ds(p * _NCODES + cb * L, L)] = packed

        def run_groups(buf):
            @plsc.parallel_loop(0, ch, step=L, unroll=4)
            def _(base):
                bits = [
                    in_v[pl.ds((buf * _W + j) * ch + base, L)] for j in range(_W)
                ]
                code = bits[0]
                for j in range(1, _W):
                    code = code * 2 + bits[j]
                for p in range(3):
                    packed = plsc.load_gather(plut_v, [code + p * _NCODES])
                    a, b = plsc.unpack(
                        plsc.bitcast(packed, jnp.bfloat16),
                        format=plsc.PackFormat.INTERLEAVED,
                    )
                    out_v[pl.ds((buf * _W + 2 * p) * ch + base, L)] = a
                    out_v[pl.ds((buf * _W + 2 * p + 1) * ch + base, L)] = b
                out_v[pl.ds((buf * _W + 6) * ch + base, L)] = plsc.load_gather(
                    lut_v, [code + 6 * _NCODES]
                )

        for ci in range(nchunk):
            buf = ci % 2
            wait_all(in_copies(ci, buf))
            if ci >= 2:
                wait_all(out_copies(ci - 2, buf))
            run_groups(buf)
            start_all(out_copies(ci, buf))
            if ci + 2 < nchunk:
                start_all(in_copies(ci + 2, buf))
        wait_all(out_copies(nchunk - 2, nchunk % 2))
        wait_all(out_copies(nchunk - 1, (nchunk - 1) % 2))

    return sc_call


def kernel(harddecision, C):
    B = harddecision.shape[0]
    hd_planes = harddecision.transpose(2, 1, 0).reshape(_W * B)
    out_planes = _build_sc_call(B)(hd_planes, C.reshape(-1))
    return out_planes.reshape(_W, 1, B).transpose(2, 1, 0)


# speedup vs baseline: 1.0092x; 1.0092x over previous
import functools

import jax
import jax.numpy as jnp
from jax import lax
from jax.experimental import pallas as pl
from jax.experimental.pallas import tpu as pltpu
from jax.experimental.pallas import tpu_sc as plsc

_W = 7
_K = 16
_NCODES = 1 << _W


@functools.lru_cache(maxsize=None)
def _build_sc_call(B: int):
    info = plsc.get_sparse_core_info()
    NC, NS, L = info.num_cores, info.num_subcores, info.num_lanes
    NW = NC * NS
    assert B % (NW * L) == 0, B
    s_per_w = B // NW

    ch = 4096
    while s_per_w % ch:
        ch //= 2
    nchunk = s_per_w // ch

    mesh = plsc.VectorSubcoreMesh(core_axis_name="c", subcore_axis_name="s")

    @functools.partial(
        pl.kernel,
        mesh=mesh,
        out_type=jax.ShapeDtypeStruct((_W * B,), jnp.float32),
        compiler_params=pltpu.CompilerParams(needs_layout_passes=False),
        scratch_types=[
            pltpu.VMEM((128,), jnp.float32),
            pltpu.VMEM((_W * _NCODES,), jnp.float32),
            pltpu.VMEM((3 * _NCODES,), jnp.int32),
            pltpu.VMEM((2 * _W * ch,), jnp.int32),
            pltpu.VMEM((2 * _W * ch,), jnp.float32),
            pltpu.SemaphoreType.DMA((2,)),
            pltpu.SemaphoreType.DMA((2,)),
        ],
    )
    def sc_call(hd_hbm, c_hbm, out_hbm, c_v, lut_v, plut_v, in_v, out_v, sin, sout):
        iota = lax.iota(jnp.int32, L)
        lane_lt7 = iota < _W
        col7 = jnp.minimum(iota, _W - 1)

        wid = lax.axis_index("s") * NC + lax.axis_index("c")
        tile_base = wid * s_per_w

        def in_copies(ci, buf):
            off = pl.multiple_of(tile_base + ci * ch, 8)
            return [
                pltpu.make_async_copy(
                    hd_hbm.at[pl.ds(j * B + off, ch)],
                    in_v.at[pl.ds((buf * _W + j) * ch, ch)],
                    sin.at[buf],
                )
                for j in range(_W)
            ]

        def out_copies(ci, buf):
            off = pl.multiple_of(tile_base + ci * ch, 8)
            return [
                pltpu.make_async_copy(
                    out_v.at[pl.ds((buf * _W + j) * ch, ch)],
                    out_hbm.at[pl.ds(j * B + off, ch)],
                    sout.at[buf],
                )
                for j in range(_W)
            ]

        def start_all(copies):
            for cp in copies:
                cp.start()

        def wait_in(buf):
            pltpu.make_async_copy(
                hd_hbm.at[pl.ds(0, _W * ch)],
                in_v.at[pl.ds(buf * _W * ch, _W * ch)],
                sin.at[buf],
            ).wait()

        def wait_out(buf):
            pltpu.make_async_copy(
                out_v.at[pl.ds(buf * _W * ch, _W * ch)],
                out_hbm.at[pl.ds(0, _W * ch)],
                sout.at[buf],
            ).wait()

        start_all(in_copies(0, 0))
        if nchunk > 1:
            start_all(in_copies(1, 1))

        pltpu.sync_copy(c_hbm, c_v.at[pl.ds(0, _K * _W)])

        ccols = [
            plsc.load_gather(c_v, [iota * _W + j]).astype(jnp.int32)
            for j in range(_W)
        ]

        def build_body(w, carry):
            dist = jnp.zeros((L,), jnp.int32)
            for j in range(_W):
                bit = (w >> (_W - 1 - j)) & 1
                dist = dist + jnp.where(ccols[j] == bit, 1, 0).astype(jnp.int32)
            score = dist * _K + (_K - 1 - iota)
            best = jnp.max(score)
            bestk = (_K - 1) - (best & (_K - 1))
            row = plsc.load_gather(c_v, [bestk * _W + col7], mask=lane_lt7)
            plsc.store_scatter(lut_v, [col7 * _NCODES + w], row, mask=lane_lt7)
            return carry

        lax.fori_loop(0, _NCODES, build_body, 0)

        for p in range(3):
            for cb in range(_NCODES // L):
                a = lut_v[pl.ds((2 * p) * _NCODES + cb * L, L)]
                b = lut_v[pl.ds((2 * p + 1) * _NCODES + cb * L, L)]
                packed = plsc.bitcast(
                    plsc.pack(a, b, format=plsc.PackFormat.INTERLEAVED),
                    jnp.int32,
                )
                plut_v[pl.ds(p * _NCODES + cb * L, L)] = packed

        def run_groups(buf):
            @plsc.parallel_loop(0, ch, step=L, unroll=4)
            def _(base):
                bits = [
                    in_v[pl.ds((buf * _W + j) * ch + base, L)] for j in range(_W)
                ]
                code = bits[0]
                for j in range(1, _W):
                    code = code * 2 + bits[j]
                for p in range(3):
                    packed = plsc.load_gather(plut_v, [code + p * _NCODES])
                    a, b = plsc.unpack(
                        plsc.bitcast(packed, jnp.bfloat16),
                        format=plsc.PackFormat.INTERLEAVED,
                    )
                    out_v[pl.ds((buf * _W + 2 * p) * ch + base, L)] = a
                    out_v[pl.ds((buf * _W + 2 * p + 1) * ch + base, L)] = b
                out_v[pl.ds((buf * _W + 6) * ch + base, L)] = plsc.load_gather(
                    lut_v, [code + 6 * _NCODES]
                )

        for ci in range(nchunk):
            buf = ci % 2
            wait_in(buf)
            if ci >= 2:
                wait_out(buf)
            run_groups(buf)
            start_all(out_copies(ci, buf))
            if ci + 2 < nchunk:
                start_all(in_copies(ci + 2, buf))
        wait_out(nchunk % 2)
        wait_out((nchunk - 1) % 2)

    return sc_call


def kernel(harddecision, C):
    B = harddecision.shape[0]
    hd_planes = harddecision.transpose(2, 1, 0).reshape(_W * B)
    out_planes = _build_sc_call(B)(hd_planes, C.reshape(-1))
    return out_planes.reshape(_W, 1, B).transpose(2, 1, 0)
